# baseline (speedup 1.0000x reference)
import math
import functools
import jax
import jax.numpy as jnp
from jax import lax
from jax.experimental import pallas as pl
from jax.experimental.pallas import tpu as pltpu
from jax.experimental.pallas import tpu_sc as plsc

B, N, NU, D = 32, 4096, 2048, 128
NUP = NU + 128
BB = 1
SQRT_D = math.sqrt(float(D))
CLIP = 10.0
I32MAX = 2**31 - 1
HI = jax.lax.Precision.HIGHEST

_SC_MESH = plsc.VectorSubcoreMesh(core_axis_name="c", subcore_axis_name="s")
L = 16


@functools.partial(
    pl.kernel,
    out_type=[jax.ShapeDtypeStruct((B, NUP), jnp.float32),
              jax.ShapeDtypeStruct((B, NUP), jnp.float32)],
    mesh=_SC_MESH,
    compiler_params=pltpu.CompilerParams(needs_layout_passes=False),
    scratch_types=[pltpu.VMEM((N * 2,), jnp.float32),
                   pltpu.VMEM((NUP,), jnp.int32),
                   pltpu.VMEM((NUP,), jnp.float32),
                   pltpu.VMEM((NUP,), jnp.float32)],
)
def _sc_gather(problems_hbm, idxp_hbm, gx_hbm, gy_hbm, pv, iv, xv, yv):
    b = lax.axis_index("s") * 2 + lax.axis_index("c")
    pltpu.sync_copy(problems_hbm.at[b], pv)
    pltpu.sync_copy(idxp_hbm.at[b], iv)

    def body(t, carry):
        ivec = iv[pl.ds(t * L, L)] * 2
        xv[pl.ds(t * L, L)] = plsc.load_gather(pv, [ivec])
        yv[pl.ds(t * L, L)] = plsc.load_gather(pv, [ivec + 1])
        return carry

    lax.fori_loop(0, NUP // L, body, 0)
    pltpu.sync_copy(xv, gx_hbm.at[b])
    pltpu.sync_copy(yv, gy_hbm.at[b])


@functools.partial(
    pl.kernel,
    out_type=jax.ShapeDtypeStruct((B, N), jnp.float32),
    mesh=_SC_MESH,
    compiler_params=pltpu.CompilerParams(needs_layout_passes=False),
    scratch_types=[pltpu.VMEM((NU + L,), jnp.int32),
                   pltpu.VMEM((NU,), jnp.float32),
                   pltpu.VMEM((N,), jnp.float32)],
)
def _sc_scatter(probs_hbm, idx_hbm, upper_hbm, iv, pv, ov):
    b = lax.axis_index("s") * 2 + lax.axis_index("c")
    pltpu.sync_copy(idx_hbm.at[b], iv.at[pl.ds(0, NU)])
    iv[pl.ds(NU, L)] = jnp.full((L,), -1, jnp.int32)
    pltpu.sync_copy(probs_hbm.at[b], pv)
    zf = jnp.zeros((L,), jnp.float32)
    lane = lax.iota(jnp.int32, L)

    def zbody(t, carry):
        ov[pl.ds(t * L, L)] = zf
        return carry

    lax.fori_loop(0, N // L, zbody, 0)

    def body(t, carry):
        cur = iv[pl.ds(t * L, L)]
        nxt = plsc.load_gather(iv, [lane + (t * L + 1)])
        plsc.store_scatter(ov, [cur], pv[pl.ds(t * L, L)], mask=cur != nxt)
        return carry

    lax.fori_loop(0, NU // L, body, 0)
    pltpu.sync_copy(ov, upper_hbm.at[b])


def _tc_body(gx_ref, gy_ref, cd_ref, nm_ref, idx_ref, We_ref, be_ref,
             Wqf_ref, Wql_ref, Wk_ref, Wv_ref, ls_ref, aa_ref, ac_ref,
             probs_ref, ts_ref, ss_ref):
    We = We_ref[...]
    bec = be_ref[...]
    Wk = Wk_ref[...]
    Wv = Wv_ref[...]
    Wq = Wqf_ref[...] + Wql_ref[...]
    lsv = ls_ref[...]
    aav = aa_ref[...]
    acv = ac_ref[...]

    We3 = jnp.concatenate([We, bec], axis=1)
    Wke3 = jnp.dot(Wk, We3, precision=HI, preferred_element_type=jnp.float32)
    Wve3 = jnp.dot(Wv, We3, precision=HI, preferred_element_type=jnp.float32)

    ones_row = jnp.ones((1, NUP), jnp.float32)
    for i in range(BB):
        G = jnp.concatenate([gx_ref[i], gy_ref[i], ones_row], axis=0)
        cd = cd_ref[i]
        nm = nm_ref[i]

        kT = jnp.dot(Wke3, G, precision=HI, preferred_element_type=jnp.float32)
        vT = jnp.dot(Wve3, G, precision=HI, preferred_element_type=jnp.float32)
        ecol = jnp.dot(We3, G[:, NU:NU + 1], precision=HI,
                       preferred_element_type=jnp.float32)
        q = jnp.dot(Wq, ecol, precision=HI, preferred_element_type=jnp.float32)

        ekT = jnp.exp(kT[:, :NU])
        evT = ekT * vT[:, :NU]
        eb = jnp.exp(nm - (lsv * aav) * cd)
        num = lax.dot_general(evT, eb, (((1,), (1,)), ((), ())), precision=HI,
                              preferred_element_type=jnp.float32)
        den = lax.dot_general(ekT, eb, (((1,), (1,)), ((), ())), precision=HI,
                              preferred_element_type=jnp.float32)
        aafm = jax.nn.sigmoid(q) * num / den
        wa = lax.dot_general(We3, aafm, (((0,), (0,)), ((), ())), precision=HI,
                             preferred_element_type=jnp.float32)
        score = lax.dot_general(wa, G[:, :NU], (((0,), (0,)), ((), ())),
                                precision=HI,
                                preferred_element_type=jnp.float32)
        score = score * (1.0 / SQRT_D) - (lsv * acv) * cd
        score = CLIP * jnp.tanh(score) + nm
        pm = jnp.max(score, axis=1, keepdims=True)
        p = jnp.exp(score - pm)
        s = jnp.sum(p, axis=1, keepdims=True)
        probs = p / s
        probs_ref[i] = probs

        mx = jnp.max(probs, axis=1, keepdims=True)
        idxv = idx_ref[i]
        tsel = jnp.min(jnp.where(probs == mx, idxv, I32MAX), axis=1, keepdims=True)
        ts_ref[i] = tsel
        ss_ref[i] = mx


def _tc_call(gx, gy, cd, nm, idx, We, bec, Wqf, Wql, Wk, Wv, ls, aa, ac):
    rep = lambda shape: pl.BlockSpec(shape, lambda b: (0,) * len(shape))
    row = lambda k: pl.BlockSpec((BB, 1, k), lambda b: (b, 0, 0))
    return pl.pallas_call(
        _tc_body,
        grid=(B // BB,),
        in_specs=[
            row(NUP), row(NUP), row(NU), row(NU), row(NU),
            rep((D, 2)), rep((D, 1)), rep((D, D)), rep((D, D)),
            rep((D, D)), rep((D, D)), rep((1, 1)), rep((1, 1)), rep((1, 1)),
        ],
        out_specs=[row(NU), pl.BlockSpec((BB, 1, 1), lambda b: (b, 0, 0)),
                   pl.BlockSpec((BB, 1, 1), lambda b: (b, 0, 0))],
        out_shape=[
            jax.ShapeDtypeStruct((B, 1, NU), jnp.float32),
            jax.ShapeDtypeStruct((B, 1, 1), jnp.int32),
            jax.ShapeDtypeStruct((B, 1, 1), jnp.float32),
        ],
    )(gx, gy, cd, nm, idx, We, bec, Wqf, Wql, Wk, Wv, ls, aa, ac)


def kernel(problems, current_node, unvisited_index, cur_dist, ninf_mask, log_scale, W_embed, b_embed, Wq_first, Wq_last, Wk, Wv, alpha_attn, alpha_com):
    idx = unvisited_index
    idxp = jnp.concatenate(
        [idx, jnp.broadcast_to(current_node[:, None], (B, 128))], axis=1)

    gxf, gyf = _sc_gather(problems.reshape(B, N * 2), idxp)

    probs3, ts3, ss3 = _tc_call(
        gxf.reshape(B, 1, NUP), gyf.reshape(B, 1, NUP),
        cur_dist, ninf_mask, idx.reshape(B, 1, NU),
        W_embed, b_embed.reshape(D, 1), Wq_first, Wq_last, Wk, Wv,
        log_scale.reshape(1, 1), alpha_attn.reshape(1, 1), alpha_com.reshape(1, 1))
    probs = probs3[:, 0, :]
    tsel = ts3[:, 0, 0]
    ssel = ss3[:, 0, 0]

    upper = _sc_scatter(probs, idx)
    return (upper, tsel, ssel)


# speedup vs baseline: 1.7748x; 1.7748x over previous
import math
import functools
import jax
import jax.numpy as jnp
from jax import lax
from jax.experimental import pallas as pl
from jax.experimental.pallas import tpu as pltpu
from jax.experimental.pallas import tpu_sc as plsc

B, N, NU, D = 32, 4096, 2048, 128
NUP = NU + 128
BB = 4
SQRT_D = math.sqrt(float(D))
CLIP = 10.0
I32MAX = 2**31 - 1
HI = jax.lax.Precision.HIGHEST

_SC_MESH = plsc.VectorSubcoreMesh(core_axis_name="c", subcore_axis_name="s")
L = 16


@functools.partial(
    pl.kernel,
    out_type=[jax.ShapeDtypeStruct((B, NUP), jnp.float32),
              jax.ShapeDtypeStruct((B, NUP), jnp.float32)],
    mesh=_SC_MESH,
    compiler_params=pltpu.CompilerParams(needs_layout_passes=False),
    scratch_types=[pltpu.VMEM((N * 2,), jnp.float32),
                   pltpu.VMEM((NUP,), jnp.int32),
                   pltpu.VMEM((NUP,), jnp.float32),
                   pltpu.VMEM((NUP,), jnp.float32)],
)
def _sc_gather(problems_hbm, idxp_hbm, gx_hbm, gy_hbm, pv, iv, xv, yv):
    b = lax.axis_index("s") * 2 + lax.axis_index("c")
    pltpu.sync_copy(problems_hbm.at[b], pv)
    pltpu.sync_copy(idxp_hbm.at[b], iv)

    def body(t, carry):
        ivec = iv[pl.ds(t * L, L)] * 2
        xv[pl.ds(t * L, L)] = plsc.load_gather(pv, [ivec])
        yv[pl.ds(t * L, L)] = plsc.load_gather(pv, [ivec + 1])
        return carry

    lax.fori_loop(0, NUP // L, body, 0)
    pltpu.sync_copy(xv, gx_hbm.at[b])
    pltpu.sync_copy(yv, gy_hbm.at[b])


@functools.partial(
    pl.kernel,
    out_type=jax.ShapeDtypeStruct((B, N), jnp.float32),
    mesh=_SC_MESH,
    compiler_params=pltpu.CompilerParams(needs_layout_passes=False),
    scratch_types=[pltpu.VMEM((NU + L,), jnp.int32),
                   pltpu.VMEM((NU,), jnp.float32),
                   pltpu.VMEM((N,), jnp.float32)],
)
def _sc_scatter(probs_hbm, idx_hbm, upper_hbm, iv, pv, ov):
    b = lax.axis_index("s") * 2 + lax.axis_index("c")
    pltpu.sync_copy(idx_hbm.at[b], iv.at[pl.ds(0, NU)])
    iv[pl.ds(NU, L)] = jnp.full((L,), -1, jnp.int32)
    pltpu.sync_copy(probs_hbm.at[b], pv)
    zf = jnp.zeros((L,), jnp.float32)
    lane = lax.iota(jnp.int32, L)

    def zbody(t, carry):
        ov[pl.ds(t * L, L)] = zf
        return carry

    lax.fori_loop(0, N // L, zbody, 0)

    def body(t, carry):
        cur = iv[pl.ds(t * L, L)]
        nxt = plsc.load_gather(iv, [lane + (t * L + 1)])
        plsc.store_scatter(ov, [cur], pv[pl.ds(t * L, L)], mask=cur != nxt)
        return carry

    lax.fori_loop(0, NU // L, body, 0)
    pltpu.sync_copy(ov, upper_hbm.at[b])


def _tc_body(gx_ref, gy_ref, cd_ref, nm_ref, idx_ref, We_ref, be_ref,
             Wqf_ref, Wql_ref, Wk_ref, Wv_ref, ls_ref, aa_ref, ac_ref,
             probs_ref, ts_ref, ss_ref):
    We = We_ref[...]
    bec = be_ref[...]
    Wk = Wk_ref[...]
    Wv = Wv_ref[...]
    Wq = Wqf_ref[...] + Wql_ref[...]
    lsv = ls_ref[...]
    aav = aa_ref[...]
    acv = ac_ref[...]

    Wke = jnp.dot(Wk, We, precision=HI, preferred_element_type=jnp.float32)
    bk = jnp.dot(Wk, bec, precision=HI, preferred_element_type=jnp.float32)
    Wve = jnp.dot(Wv, We, precision=HI, preferred_element_type=jnp.float32)
    bv = jnp.dot(Wv, bec, precision=HI, preferred_element_type=jnp.float32)
    Wx = We[:, 0:1]
    Wy = We[:, 1:2]

    for i in range(BB):
        gxr = gx_ref[i]
        gyr = gy_ref[i]
        cd = cd_ref[i]
        nm = nm_ref[i]

        eT = Wx * gxr + Wy * gyr + bec
        kT = Wke[:, 0:1] * gxr + Wke[:, 1:2] * gyr + bk
        vT = Wve[:, 0:1] * gxr + Wve[:, 1:2] * gyr + bv

        ecol = eT[:, NU:NU + 1]
        q = jnp.dot(Wq, ecol, precision=HI, preferred_element_type=jnp.float32)

        ekT = jnp.exp(kT[:, :NU])
        evT = ekT * vT[:, :NU]
        eb = jnp.exp(nm - (lsv * aav) * cd)
        num = lax.dot_general(evT, eb, (((1,), (1,)), ((), ())), precision=HI,
                              preferred_element_type=jnp.float32)
        den = lax.dot_general(ekT, eb, (((1,), (1,)), ((), ())), precision=HI,
                              preferred_element_type=jnp.float32)
        aafm = jax.nn.sigmoid(q) * num / den
        score = lax.dot_general(aafm, eT[:, :NU], (((0,), (0,)), ((), ())),
                                precision=HI,
                                preferred_element_type=jnp.float32)
        score = score * (1.0 / SQRT_D) - (lsv * acv) * cd
        score = CLIP * jnp.tanh(score) + nm
        pm = jnp.max(score, axis=1, keepdims=True)
        p = jnp.exp(score - pm)
        s = jnp.sum(p, axis=1, keepdims=True)
        probs = p / s
        probs_ref[i] = probs

        mx = jnp.max(probs, axis=1, keepdims=True)
        idxv = idx_ref[i]
        tsel = jnp.min(jnp.where(probs == mx, idxv, I32MAX), axis=1, keepdims=True)
        ts_ref[i] = tsel
        ss_ref[i] = mx


def _tc_call(gx, gy, cd, nm, idx, We, bec, Wqf, Wql, Wk, Wv, ls, aa, ac):
    rep = lambda shape: pl.BlockSpec(shape, lambda b: (0,) * len(shape))
    row = lambda k: pl.BlockSpec((BB, 1, k), lambda b: (b, 0, 0))
    return pl.pallas_call(
        _tc_body,
        grid=(B // BB,),
        in_specs=[
            row(NUP), row(NUP), row(NU), row(NU), row(NU),
            rep((D, 2)), rep((D, 1)), rep((D, D)), rep((D, D)),
            rep((D, D)), rep((D, D)), rep((1, 1)), rep((1, 1)), rep((1, 1)),
        ],
        out_specs=[row(NU), pl.BlockSpec((BB, 1, 1), lambda b: (b, 0, 0)),
                   pl.BlockSpec((BB, 1, 1), lambda b: (b, 0, 0))],
        out_shape=[
            jax.ShapeDtypeStruct((B, 1, NU), jnp.float32),
            jax.ShapeDtypeStruct((B, 1, 1), jnp.int32),
            jax.ShapeDtypeStruct((B, 1, 1), jnp.float32),
        ],
    )(gx, gy, cd, nm, idx, We, bec, Wqf, Wql, Wk, Wv, ls, aa, ac)


def kernel(problems, current_node, unvisited_index, cur_dist, ninf_mask, log_scale, W_embed, b_embed, Wq_first, Wq_last, Wk, Wv, alpha_attn, alpha_com):
    idx = unvisited_index
    idxp = jnp.concatenate(
        [idx, jnp.broadcast_to(current_node[:, None], (B, 128))], axis=1)

    gxf, gyf = _sc_gather(problems.reshape(B, N * 2), idxp)

    probs3, ts3, ss3 = _tc_call(
        gxf.reshape(B, 1, NUP), gyf.reshape(B, 1, NUP),
        cur_dist, ninf_mask, idx.reshape(B, 1, NU),
        W_embed, b_embed.reshape(D, 1), Wq_first, Wq_last, Wk, Wv,
        log_scale.reshape(1, 1), alpha_attn.reshape(1, 1), alpha_com.reshape(1, 1))
    probs = probs3[:, 0, :]
    tsel = ts3[:, 0, 0]
    ssel = ss3[:, 0, 0]

    upper = _sc_scatter(probs, idx)
    return (upper, tsel, ssel)


# speedup vs baseline: 1.8849x; 1.0620x over previous
import math
import functools
import jax
import jax.numpy as jnp
from jax import lax
from jax.experimental import pallas as pl
from jax.experimental.pallas import tpu as pltpu
from jax.experimental.pallas import tpu_sc as plsc

B, N, NU, D = 32, 4096, 2048, 128
NUP = NU + 128
BB = 8
SQRT_D = math.sqrt(float(D))
CLIP = 10.0
I32MAX = 2**31 - 1
HI = jax.lax.Precision.HIGHEST

_SC_MESH = plsc.VectorSubcoreMesh(core_axis_name="c", subcore_axis_name="s")
L = 16


@functools.partial(
    pl.kernel,
    out_type=[jax.ShapeDtypeStruct((B, NUP), jnp.float32),
              jax.ShapeDtypeStruct((B, NUP), jnp.float32)],
    mesh=_SC_MESH,
    compiler_params=pltpu.CompilerParams(needs_layout_passes=False),
    scratch_types=[pltpu.VMEM((N * 2,), jnp.float32),
                   pltpu.VMEM((NUP,), jnp.int32),
                   pltpu.VMEM((NUP,), jnp.float32),
                   pltpu.VMEM((NUP,), jnp.float32)],
)
def _sc_gather(problems_hbm, idxp_hbm, gx_hbm, gy_hbm, pv, iv, xv, yv):
    b = lax.axis_index("s") * 2 + lax.axis_index("c")
    pltpu.sync_copy(problems_hbm.at[b], pv)
    pltpu.sync_copy(idxp_hbm.at[b], iv)

    def body(t, carry):
        ivec = iv[pl.ds(t * L, L)] * 2
        xv[pl.ds(t * L, L)] = plsc.load_gather(pv, [ivec])
        yv[pl.ds(t * L, L)] = plsc.load_gather(pv, [ivec + 1])
        return carry

    lax.fori_loop(0, NUP // L, body, 0)
    pltpu.sync_copy(xv, gx_hbm.at[b])
    pltpu.sync_copy(yv, gy_hbm.at[b])


@functools.partial(
    pl.kernel,
    out_type=jax.ShapeDtypeStruct((B, N), jnp.float32),
    mesh=_SC_MESH,
    compiler_params=pltpu.CompilerParams(needs_layout_passes=False),
    scratch_types=[pltpu.VMEM((NU + L,), jnp.int32),
                   pltpu.VMEM((NU,), jnp.float32),
                   pltpu.VMEM((N,), jnp.float32)],
)
def _sc_scatter(probs_hbm, idx_hbm, upper_hbm, iv, pv, ov):
    b = lax.axis_index("s") * 2 + lax.axis_index("c")
    pltpu.sync_copy(idx_hbm.at[b], iv.at[pl.ds(0, NU)])
    iv[pl.ds(NU, L)] = jnp.full((L,), -1, jnp.int32)
    pltpu.sync_copy(probs_hbm.at[b], pv)
    zf = jnp.zeros((L,), jnp.float32)
    lane = lax.iota(jnp.int32, L)

    def zbody(t, carry):
        ov[pl.ds(t * L, L)] = zf
        return carry

    lax.fori_loop(0, N // L, zbody, 0)

    def body(t, carry):
        cur = iv[pl.ds(t * L, L)]
        nxt = plsc.load_gather(iv, [lane + (t * L + 1)])
        plsc.store_scatter(ov, [cur], pv[pl.ds(t * L, L)], mask=cur != nxt)
        return carry

    lax.fori_loop(0, NU // L, body, 0)
    pltpu.sync_copy(ov, upper_hbm.at[b])


def _tc_body(gx_ref, gy_ref, cd_ref, nm_ref, idx_ref, We_ref, be_ref,
             Wqf_ref, Wql_ref, Wk_ref, Wv_ref, ls_ref, aa_ref, ac_ref,
             probs_ref, ts_ref, ss_ref):
    We = We_ref[...]
    bec = be_ref[...]
    Wk = Wk_ref[...]
    Wv = Wv_ref[...]
    Wq = Wqf_ref[...] + Wql_ref[...]
    lsv = ls_ref[...]
    aav = aa_ref[...]
    acv = ac_ref[...]

    Wke = jnp.dot(Wk, We, precision=HI, preferred_element_type=jnp.float32)
    bk = jnp.dot(Wk, bec, precision=HI, preferred_element_type=jnp.float32)
    Wve = jnp.dot(Wv, We, precision=HI, preferred_element_type=jnp.float32)
    bv = jnp.dot(Wv, bec, precision=HI, preferred_element_type=jnp.float32)
    Wx = We[:, 0:1]
    Wy = We[:, 1:2]

    for i in range(BB):
        gxr = gx_ref[i]
        gyr = gy_ref[i]
        cd = cd_ref[i]
        nm = nm_ref[i]

        eT = Wx * gxr + Wy * gyr + bec
        kT = Wke[:, 0:1] * gxr + Wke[:, 1:2] * gyr + bk
        vT = Wve[:, 0:1] * gxr + Wve[:, 1:2] * gyr + bv

        ecol = eT[:, NU:NU + 1]
        q = jnp.dot(Wq, ecol, precision=HI, preferred_element_type=jnp.float32)

        ekT = jnp.exp(kT[:, :NU])
        evT = ekT * vT[:, :NU]
        eb = jnp.exp(nm - (lsv * aav) * cd)
        num = lax.dot_general(evT, eb, (((1,), (1,)), ((), ())), precision=HI,
                              preferred_element_type=jnp.float32)
        den = lax.dot_general(ekT, eb, (((1,), (1,)), ((), ())), precision=HI,
                              preferred_element_type=jnp.float32)
        aafm = jax.nn.sigmoid(q) * num / den
        score = lax.dot_general(aafm, eT[:, :NU], (((0,), (0,)), ((), ())),
                                precision=HI,
                                preferred_element_type=jnp.float32)
        score = score * (1.0 / SQRT_D) - (lsv * acv) * cd
        score = CLIP * jnp.tanh(score) + nm
        pm = jnp.max(score, axis=1, keepdims=True)
        p = jnp.exp(score - pm)
        s = jnp.sum(p, axis=1, keepdims=True)
        probs = p / s
        probs_ref[i] = probs

        mx = jnp.max(probs, axis=1, keepdims=True)
        idxv = idx_ref[i]
        tsel = jnp.min(jnp.where(probs == mx, idxv, I32MAX), axis=1, keepdims=True)
        ts_ref[i] = tsel
        ss_ref[i] = mx


def _tc_call(gx, gy, cd, nm, idx, We, bec, Wqf, Wql, Wk, Wv, ls, aa, ac):
    rep = lambda shape: pl.BlockSpec(shape, lambda b: (0,) * len(shape))
    row = lambda k: pl.BlockSpec((BB, 1, k), lambda b: (b, 0, 0))
    return pl.pallas_call(
        _tc_body,
        grid=(B // BB,),
        in_specs=[
            row(NUP), row(NUP), row(NU), row(NU), row(NU),
            rep((D, 2)), rep((D, 1)), rep((D, D)), rep((D, D)),
            rep((D, D)), rep((D, D)), rep((1, 1)), rep((1, 1)), rep((1, 1)),
        ],
        out_specs=[row(NU), pl.BlockSpec((BB, 1, 1), lambda b: (b, 0, 0)),
                   pl.BlockSpec((BB, 1, 1), lambda b: (b, 0, 0))],
        out_shape=[
            jax.ShapeDtypeStruct((B, 1, NU), jnp.float32),
            jax.ShapeDtypeStruct((B, 1, 1), jnp.int32),
            jax.ShapeDtypeStruct((B, 1, 1), jnp.float32),
        ],
    )(gx, gy, cd, nm, idx, We, bec, Wqf, Wql, Wk, Wv, ls, aa, ac)


def kernel(problems, current_node, unvisited_index, cur_dist, ninf_mask, log_scale, W_embed, b_embed, Wq_first, Wq_last, Wk, Wv, alpha_attn, alpha_com):
    idx = unvisited_index
    idxp = jnp.concatenate(
        [idx, jnp.broadcast_to(current_node[:, None], (B, 128))], axis=1)

    gxf, gyf = _sc_gather(problems.reshape(B, N * 2), idxp)

    probs3, ts3, ss3 = _tc_call(
        gxf.reshape(B, 1, NUP), gyf.reshape(B, 1, NUP),
        cur_dist, ninf_mask, idx.reshape(B, 1, NU),
        W_embed, b_embed.reshape(D, 1), Wq_first, Wq_last, Wk, Wv,
        log_scale.reshape(1, 1), alpha_attn.reshape(1, 1), alpha_com.reshape(1, 1))
    probs = probs3[:, 0, :]
    tsel = ts3[:, 0, 0]
    ssel = ss3[:, 0, 0]

    upper = _sc_scatter(probs, idx)
    return (upper, tsel, ssel)


# speedup vs baseline: 1.9521x; 1.0357x over previous
import math
import functools
import jax
import jax.numpy as jnp
from jax import lax
from jax.experimental import pallas as pl
from jax.experimental.pallas import tpu as pltpu
from jax.experimental.pallas import tpu_sc as plsc

B, N, NU, D = 32, 4096, 2048, 128
NUP = NU + 128
BB = 16
SQRT_D = math.sqrt(float(D))
CLIP = 10.0
I32MAX = 2**31 - 1
HI = jax.lax.Precision.HIGHEST

_SC_MESH = plsc.VectorSubcoreMesh(core_axis_name="c", subcore_axis_name="s")
L = 16


@functools.partial(
    pl.kernel,
    out_type=[jax.ShapeDtypeStruct((B, NUP), jnp.float32),
              jax.ShapeDtypeStruct((B, NUP), jnp.float32)],
    mesh=_SC_MESH,
    compiler_params=pltpu.CompilerParams(needs_layout_passes=False),
    scratch_types=[pltpu.VMEM((N * 2,), jnp.float32),
                   pltpu.VMEM((NUP,), jnp.int32),
                   pltpu.VMEM((NUP,), jnp.float32),
                   pltpu.VMEM((NUP,), jnp.float32)],
)
def _sc_gather(problems_hbm, idxp_hbm, gx_hbm, gy_hbm, pv, iv, xv, yv):
    b = lax.axis_index("s") * 2 + lax.axis_index("c")
    pltpu.sync_copy(problems_hbm.at[b], pv)
    pltpu.sync_copy(idxp_hbm.at[b], iv)

    def body(t, carry):
        ivec = iv[pl.ds(t * L, L)] * 2
        xv[pl.ds(t * L, L)] = plsc.load_gather(pv, [ivec])
        yv[pl.ds(t * L, L)] = plsc.load_gather(pv, [ivec + 1])
        return carry

    lax.fori_loop(0, NUP // L, body, 0)
    pltpu.sync_copy(xv, gx_hbm.at[b])
    pltpu.sync_copy(yv, gy_hbm.at[b])


@functools.partial(
    pl.kernel,
    out_type=jax.ShapeDtypeStruct((B, N), jnp.float32),
    mesh=_SC_MESH,
    compiler_params=pltpu.CompilerParams(needs_layout_passes=False),
    scratch_types=[pltpu.VMEM((NU + L,), jnp.int32),
                   pltpu.VMEM((NU,), jnp.float32),
                   pltpu.VMEM((N,), jnp.float32)],
)
def _sc_scatter(probs_hbm, idx_hbm, upper_hbm, iv, pv, ov):
    b = lax.axis_index("s") * 2 + lax.axis_index("c")
    pltpu.sync_copy(idx_hbm.at[b], iv.at[pl.ds(0, NU)])
    iv[pl.ds(NU, L)] = jnp.full((L,), -1, jnp.int32)
    pltpu.sync_copy(probs_hbm.at[b], pv)
    zf = jnp.zeros((L,), jnp.float32)
    lane = lax.iota(jnp.int32, L)

    def zbody(t, carry):
        ov[pl.ds(t * L, L)] = zf
        return carry

    lax.fori_loop(0, N // L, zbody, 0)

    def body(t, carry):
        cur = iv[pl.ds(t * L, L)]
        nxt = plsc.load_gather(iv, [lane + (t * L + 1)])
        plsc.store_scatter(ov, [cur], pv[pl.ds(t * L, L)], mask=cur != nxt)
        return carry

    lax.fori_loop(0, NU // L, body, 0)
    pltpu.sync_copy(ov, upper_hbm.at[b])


def _tc_body(gx_ref, gy_ref, cd_ref, nm_ref, idx_ref, We_ref, be_ref,
             Wqf_ref, Wql_ref, Wk_ref, Wv_ref, ls_ref, aa_ref, ac_ref,
             probs_ref, ts_ref, ss_ref):
    We = We_ref[...]
    bec = be_ref[...]
    Wk = Wk_ref[...]
    Wv = Wv_ref[...]
    Wq = Wqf_ref[...] + Wql_ref[...]
    lsv = ls_ref[...]
    aav = aa_ref[...]
    acv = ac_ref[...]

    Wke = jnp.dot(Wk, We, precision=HI, preferred_element_type=jnp.float32)
    bk = jnp.dot(Wk, bec, precision=HI, preferred_element_type=jnp.float32)
    Wve = jnp.dot(Wv, We, precision=HI, preferred_element_type=jnp.float32)
    bv = jnp.dot(Wv, bec, precision=HI, preferred_element_type=jnp.float32)
    Wx = We[:, 0:1]
    Wy = We[:, 1:2]

    for i in range(BB):
        gxr = gx_ref[i]
        gyr = gy_ref[i]
        cd = cd_ref[i]
        nm = nm_ref[i]

        eT = Wx * gxr + Wy * gyr + bec
        kT = Wke[:, 0:1] * gxr + Wke[:, 1:2] * gyr + bk
        vT = Wve[:, 0:1] * gxr + Wve[:, 1:2] * gyr + bv

        ecol = eT[:, NU:NU + 1]
        q = jnp.dot(Wq, ecol, precision=HI, preferred_element_type=jnp.float32)

        ekT = jnp.exp(kT[:, :NU])
        evT = ekT * vT[:, :NU]
        eb = jnp.exp(nm - (lsv * aav) * cd)
        num = lax.dot_general(evT, eb, (((1,), (1,)), ((), ())), precision=HI,
                              preferred_element_type=jnp.float32)
        den = lax.dot_general(ekT, eb, (((1,), (1,)), ((), ())), precision=HI,
                              preferred_element_type=jnp.float32)
        aafm = jax.nn.sigmoid(q) * num / den
        score = lax.dot_general(aafm, eT[:, :NU], (((0,), (0,)), ((), ())),
                                precision=HI,
                                preferred_element_type=jnp.float32)
        score = score * (1.0 / SQRT_D) - (lsv * acv) * cd
        score = CLIP * jnp.tanh(score) + nm
        pm = jnp.max(score, axis=1, keepdims=True)
        p = jnp.exp(score - pm)
        s = jnp.sum(p, axis=1, keepdims=True)
        probs = p / s
        probs_ref[i] = probs

        mx = jnp.max(probs, axis=1, keepdims=True)
        idxv = idx_ref[i]
        tsel = jnp.min(jnp.where(probs == mx, idxv, I32MAX), axis=1, keepdims=True)
        ts_ref[i] = tsel
        ss_ref[i] = mx


def _tc_call(gx, gy, cd, nm, idx, We, bec, Wqf, Wql, Wk, Wv, ls, aa, ac):
    rep = lambda shape: pl.BlockSpec(shape, lambda b: (0,) * len(shape))
    row = lambda k: pl.BlockSpec((BB, 1, k), lambda b: (b, 0, 0))
    return pl.pallas_call(
        _tc_body,
        grid=(B // BB,),
        in_specs=[
            row(NUP), row(NUP), row(NU), row(NU), row(NU),
            rep((D, 2)), rep((D, 1)), rep((D, D)), rep((D, D)),
            rep((D, D)), rep((D, D)), rep((1, 1)), rep((1, 1)), rep((1, 1)),
        ],
        out_specs=[row(NU), pl.BlockSpec((BB, 1, 1), lambda b: (b, 0, 0)),
                   pl.BlockSpec((BB, 1, 1), lambda b: (b, 0, 0))],
        out_shape=[
            jax.ShapeDtypeStruct((B, 1, NU), jnp.float32),
            jax.ShapeDtypeStruct((B, 1, 1), jnp.int32),
            jax.ShapeDtypeStruct((B, 1, 1), jnp.float32),
        ],
    )(gx, gy, cd, nm, idx, We, bec, Wqf, Wql, Wk, Wv, ls, aa, ac)


def kernel(problems, current_node, unvisited_index, cur_dist, ninf_mask, log_scale, W_embed, b_embed, Wq_first, Wq_last, Wk, Wv, alpha_attn, alpha_com):
    idx = unvisited_index
    idxp = jnp.concatenate(
        [idx, jnp.broadcast_to(current_node[:, None], (B, 128))], axis=1)

    gxf, gyf = _sc_gather(problems.reshape(B, N * 2), idxp)

    probs3, ts3, ss3 = _tc_call(
        gxf.reshape(B, 1, NUP), gyf.reshape(B, 1, NUP),
        cur_dist, ninf_mask, idx.reshape(B, 1, NU),
        W_embed, b_embed.reshape(D, 1), Wq_first, Wq_last, Wk, Wv,
        log_scale.reshape(1, 1), alpha_attn.reshape(1, 1), alpha_com.reshape(1, 1))
    probs = probs3[:, 0, :]
    tsel = ts3[:, 0, 0]
    ssel = ss3[:, 0, 0]

    upper = _sc_scatter(probs, idx)
    return (upper, tsel, ssel)
